# baseline (speedup 1.0000x reference)
import functools

import jax
import jax.numpy as jnp
from jax import lax
from jax.experimental import pallas as pl
from jax.experimental.pallas import tpu as pltpu
from jax.experimental.pallas import tpu_sc as plsc

N = 10000
E = 320000
D = 128
R = 16
NB = 5
BN = N // NB
C = 128
K0 = 152
K1 = 8


def _hproj_body(a_ref, v_ref, feat_ref, out_ref, rw_ref):
    b = pl.program_id(0)
    r = pl.program_id(1)

    @pl.when(b == 0)
    def _():
        sel = lax.broadcasted_iota(jnp.int32, (R, 4), 0) == r
        a = jnp.where(sel, a_ref[...], 0.0).sum(axis=0)
        rw_ref[r] = (v_ref[...] * a[:, None, None]).sum(axis=0)

    out_ref[0] = jnp.dot(feat_ref[...], rw_ref[r],
                         preferred_element_type=jnp.float32)


def _hproj_tc(feat, v_b, a_rb):
    return pl.pallas_call(
        _hproj_body,
        grid=(NB, R),
        in_specs=[
            pl.BlockSpec((R, 4), lambda b, r: (0, 0)),
            pl.BlockSpec((4, D, D), lambda b, r: (0, 0, 0)),
            pl.BlockSpec((BN, D), lambda b, r: (b, 0)),
        ],
        out_specs=pl.BlockSpec((1, BN, D), lambda b, r: (r, b, 0)),
        out_shape=jax.ShapeDtypeStruct((R, N, D), jnp.float32),
        scratch_shapes=[pltpu.VMEM((R, D, D), jnp.float32)],
    )(a_rb, v_b, feat)


def _sc_body(aux_hbm, dst_hbm, norm_hbm, hproj_hbm, out_hbm,
             agg_sh, auxb, dstb, normb, rows,
             gs0, gs1, gt0, gt1, ss0, ss1, as0, as1, as2, as3,
             ds0, ds1, ds2, ds3, ns0, ns1, ns2, ns3,
             *, rows_per_sub):
    cid = lax.axis_index("c")
    sid = lax.axis_index("s")
    n_subs = lax.axis_size("s")
    kc = jnp.where(cid == 0, K0, K1)
    cb = jnp.where(cid == 0, sid * K0, n_subs * K0 + sid * K1)
    gs = (gs0, gs1)
    gt = (gt0, gt1)
    ss = (ss0, ss1)
    asem = (as0, as1, as2, as3)
    dsem = (ds0, ds1, ds2, ds3)
    nsem = (ns0, ns1, ns2, ns3)

    _ns = jax.named_scope
    zero_v = jnp.zeros((16,), jnp.float32)

    with _ns("zero_init"):
        def _zrow(i, _):
            for k in range(D // 16):
                rows[0, i, pl.ds(k * 16, 16)] = zero_v
            return 0

        lax.fori_loop(0, C, _zrow, 0)

        base_row = sid * rows_per_sub
        done = 0
        while done < rows_per_sub:
            step = min(C, rows_per_sub - done)
            pltpu.sync_copy(rows.at[0, pl.ds(0, step)],
                            agg_sh.at[pl.ds(base_row + done, step)])
            done += step

        plsc.subcore_barrier()

    def _aux_issue(g, slot):
        base = (cb + g) * C
        pltpu.async_copy(aux_hbm.at[pl.ds(base, C)],
                         auxb.at[pl.ds(slot * C, C)], asem[slot])
        pltpu.async_copy(dst_hbm.at[pl.ds(base, C)], dstb.at[slot],
                         dsem[slot])
        pltpu.async_copy(norm_hbm.at[pl.ds(base, C)], normb.at[slot],
                         nsem[slot])

    def _aux_wait(g, slot):
        base = (cb + g) * C
        pltpu.make_async_copy(aux_hbm.at[pl.ds(base, C)],
                              auxb.at[pl.ds(slot * C, C)],
                              asem[slot]).wait()
        pltpu.make_async_copy(norm_hbm.at[pl.ds(base, C)], normb.at[slot],
                              nsem[slot]).wait()

    def _dst_wait(g, slot):
        pltpu.make_async_copy(dst_hbm.at[pl.ds((cb + g) * C, C)],
                              dstb.at[slot], dsem[slot]).wait()

    H = C // 2

    def _gather_issue(slot, rp):
        pltpu.async_copy(hproj_hbm.at[auxb.at[pl.ds(slot * C, H)]],
                         rows.at[rp, pl.ds(0, H)], gs[rp])
        pltpu.async_copy(hproj_hbm.at[auxb.at[pl.ds(slot * C + H, H)]],
                         rows.at[rp, pl.ds(H, H)], gt[rp])

    def _gather_wait(slot, rp):
        pltpu.make_async_copy(hproj_hbm.at[auxb.at[pl.ds(slot * C, H)]],
                              rows.at[rp, pl.ds(0, H)], gs[rp]).wait()
        pltpu.make_async_copy(hproj_hbm.at[auxb.at[pl.ds(slot * C + H, H)]],
                              rows.at[rp, pl.ds(H, H)], gt[rp]).wait()

    def _scatter_wait(slot, rp):
        pltpu.make_async_copy(rows.at[rp], agg_sh.at[dstb.at[slot]],
                              ss[rp]).wait()

    _aux_issue(0, 0)
    _aux_issue(1, 1)
    _aux_wait(0, 0)
    _gather_issue(0, 0)

    def _iter(g, p):
        rp = p & 1
        slot, nslot, n2slot, pslot = p, (p + 1) % 4, (p + 2) % 4, (p - 1) % 4

        @pl.when(g > 0)
        def _():
            _scatter_wait(pslot, 1 - rp)

        _gather_wait(slot, rp)

        def _scale(g2, _):
            nv = normb[slot, pl.ds(g2 * 16, 16)]
            for j in range(16):
                s = nv[j]
                e = g2 * 16 + j
                for k in range(D // 16):
                    rows[rp, e, pl.ds(k * 16, 16)] = (
                        rows[rp, e, pl.ds(k * 16, 16)] * s)
            return 0

        lax.fori_loop(0, C // 16, _scale, 0)

        _dst_wait(g, slot)
        pltpu.async_copy(rows.at[rp], agg_sh.at[dstb.at[slot]], ss[rp],
                         add=True)

        @pl.when(g + 1 < kc)
        def _():
            _aux_wait(g + 1, nslot)
            _gather_issue(nslot, 1 - rp)

        @pl.when(g + 2 < kc)
        def _():
            _aux_issue(g + 2, n2slot)

    def _quad(qq, _):
        for p in range(4):
            _iter(qq * 4 + p, p)
        return 0

    with _ns("edge_loop"):
        lax.fori_loop(0, kc // 4, _quad, 0)
        _scatter_wait(3, 1)

        plsc.subcore_barrier()

    with _ns("copy_out"):
        pltpu.sync_copy(agg_sh.at[pl.ds(base_row, rows_per_sub)],
                        out_hbm.at[cid, pl.ds(base_row, rows_per_sub)])


def _sc_agg(row_idx, dst, norm, hproj_flat, n_cores, n_subs):
    total_chunks = n_subs * (K0 + K1)
    assert total_chunks * C >= E
    ep = total_chunks * C
    pad = ep - E
    row_idx = jnp.concatenate([row_idx, jnp.zeros((pad,), jnp.int32)])
    dst = jnp.concatenate([dst, jnp.zeros((pad,), jnp.int32)])
    norm = jnp.concatenate([norm, jnp.zeros((pad,), jnp.float32)])
    rows_per_sub = ((-(-N // n_subs)) + 7) // 8 * 8
    n_pad = rows_per_sub * n_subs

    mesh = plsc.VectorSubcoreMesh(core_axis_name="c", subcore_axis_name="s")
    body = functools.partial(_sc_body, rows_per_sub=rows_per_sub)
    f = pl.kernel(
        body,
        out_type=jax.ShapeDtypeStruct((n_cores, n_pad, D), jnp.float32),
        mesh=mesh,
        scratch_types=[
            pltpu.VMEM_SHARED((n_pad, D), jnp.float32),
            pltpu.VMEM((4 * C,), jnp.int32),
            pltpu.VMEM((4, C), jnp.int32),
            pltpu.VMEM((4, C), jnp.float32),
            pltpu.VMEM((2, C, D), jnp.float32),
        ] + [pltpu.SemaphoreType.DMA] * 18,
    )
    return f(row_idx, dst, norm, hproj_flat)


def _combine_body(p_ref, feat_ref, lw_ref, out_ref):
    h = p_ref[...].sum(axis=0) + jnp.dot(feat_ref[...], lw_ref[...],
                                         preferred_element_type=jnp.float32)
    out_ref[...] = jnp.maximum(h, 0.0)


def _combine_tc(partials, feat, loop_weight):
    n_cores = partials.shape[0]
    return pl.pallas_call(
        _combine_body,
        grid=(NB,),
        in_specs=[
            pl.BlockSpec((n_cores, BN, D), lambda b: (0, b, 0)),
            pl.BlockSpec((BN, D), lambda b: (b, 0)),
            pl.BlockSpec((D, D), lambda b: (0, 0)),
        ],
        out_specs=pl.BlockSpec((BN, D), lambda b: (b, 0)),
        out_shape=jax.ShapeDtypeStruct((N, D), jnp.float32),
    )(partials, feat, loop_weight)


def kernel(feat, edge_index, etype, norm, v_b, a_rb, loop_weight):
    info = plsc.get_sparse_core_info()
    n_cores, n_subs = info.num_cores, info.num_subcores

    hproj = _hproj_tc(feat, v_b, a_rb)
    hproj_flat = hproj.reshape(R * N, D)

    src = edge_index[0]
    dst = edge_index[1]
    row_idx = etype * N + src
    partials = _sc_agg(row_idx, dst, norm[:, 0], hproj_flat, n_cores, n_subs)
    return _combine_tc(partials, feat, loop_weight)


# speedup vs baseline: 1.1025x; 1.1025x over previous
import functools

import jax
import jax.numpy as jnp
from jax import lax
from jax.experimental import pallas as pl
from jax.experimental.pallas import tpu as pltpu
from jax.experimental.pallas import tpu_sc as plsc

N = 10000
E = 320000
D = 128
R = 16
NB = 5
BN = N // NB
C = 128
K0 = 136
K1 = 24


def _hproj_body(a_ref, v_ref, feat_ref, out_ref, rw_ref):
    b = pl.program_id(0)
    r = pl.program_id(1)

    @pl.when(b == 0)
    def _():
        sel = lax.broadcasted_iota(jnp.int32, (R, 4), 0) == r
        a = jnp.where(sel, a_ref[...], 0.0).sum(axis=0)
        rw_ref[r] = (v_ref[...] * a[:, None, None]).sum(axis=0)

    out_ref[0] = jnp.dot(feat_ref[...], rw_ref[r],
                         preferred_element_type=jnp.float32)


def _hproj_tc(feat, v_b, a_rb):
    return pl.pallas_call(
        _hproj_body,
        grid=(NB, R),
        in_specs=[
            pl.BlockSpec((R, 4), lambda b, r: (0, 0)),
            pl.BlockSpec((4, D, D), lambda b, r: (0, 0, 0)),
            pl.BlockSpec((BN, D), lambda b, r: (b, 0)),
        ],
        out_specs=pl.BlockSpec((1, BN, D), lambda b, r: (r, b, 0)),
        out_shape=jax.ShapeDtypeStruct((R, N, D), jnp.float32),
        scratch_shapes=[pltpu.VMEM((R, D, D), jnp.float32)],
    )(a_rb, v_b, feat)


def _sc_body(aux_hbm, dst_hbm, norm_hbm, hproj_hbm, out_hbm,
             agg_sh, auxb, dstb, normb, rows,
             gs0, gs1, gt0, gt1, ss0, ss1, as0, as1, as2, as3,
             ds0, ds1, ds2, ds3, ns0, ns1, ns2, ns3,
             *, rows_per_sub):
    cid = lax.axis_index("c")
    sid = lax.axis_index("s")
    n_subs = lax.axis_size("s")
    kc = jnp.where(cid == 0, K0, K1)
    cb = jnp.where(cid == 0, sid * K0, n_subs * K0 + sid * K1)
    gs = (gs0, gs1)
    gt = (gt0, gt1)
    ss = (ss0, ss1)
    asem = (as0, as1, as2, as3)
    dsem = (ds0, ds1, ds2, ds3)
    nsem = (ns0, ns1, ns2, ns3)

    _ns = jax.named_scope
    zero_v = jnp.zeros((16,), jnp.float32)

    with _ns("zero_init"):
        def _zrow(i, _):
            for k in range(D // 16):
                rows[0, i, pl.ds(k * 16, 16)] = zero_v
            return 0

        lax.fori_loop(0, C, _zrow, 0)

        base_row = sid * rows_per_sub
        done = 0
        while done < rows_per_sub:
            step = min(C, rows_per_sub - done)
            pltpu.sync_copy(rows.at[0, pl.ds(0, step)],
                            agg_sh.at[pl.ds(base_row + done, step)])
            done += step

        plsc.subcore_barrier()

    def _aux_issue(g, slot):
        base = (cb + g) * C
        pltpu.async_copy(aux_hbm.at[pl.ds(base, C)],
                         auxb.at[pl.ds(slot * C, C)], asem[slot])
        pltpu.async_copy(dst_hbm.at[pl.ds(base, C)], dstb.at[slot],
                         dsem[slot])
        pltpu.async_copy(norm_hbm.at[pl.ds(base, C)], normb.at[slot],
                         nsem[slot])

    def _aux_wait(g, slot):
        base = (cb + g) * C
        pltpu.make_async_copy(aux_hbm.at[pl.ds(base, C)],
                              auxb.at[pl.ds(slot * C, C)],
                              asem[slot]).wait()
        pltpu.make_async_copy(norm_hbm.at[pl.ds(base, C)], normb.at[slot],
                              nsem[slot]).wait()

    def _dst_wait(g, slot):
        pltpu.make_async_copy(dst_hbm.at[pl.ds((cb + g) * C, C)],
                              dstb.at[slot], dsem[slot]).wait()

    H = C // 2

    def _gather_issue(slot, rp):
        pltpu.async_copy(hproj_hbm.at[auxb.at[pl.ds(slot * C, H)]],
                         rows.at[rp, pl.ds(0, H)], gs[rp])
        pltpu.async_copy(hproj_hbm.at[auxb.at[pl.ds(slot * C + H, H)]],
                         rows.at[rp, pl.ds(H, H)], gt[rp])

    def _gather_wait(slot, rp):
        pltpu.make_async_copy(hproj_hbm.at[auxb.at[pl.ds(slot * C, H)]],
                              rows.at[rp, pl.ds(0, H)], gs[rp]).wait()
        pltpu.make_async_copy(hproj_hbm.at[auxb.at[pl.ds(slot * C + H, H)]],
                              rows.at[rp, pl.ds(H, H)], gt[rp]).wait()

    def _scatter_wait(slot, rp):
        pltpu.make_async_copy(rows.at[rp], agg_sh.at[dstb.at[slot]],
                              ss[rp]).wait()

    _aux_issue(0, 0)
    _aux_issue(1, 1)
    _aux_wait(0, 0)
    _gather_issue(0, 0)

    def _iter(g, p):
        rp = p & 1
        slot, nslot, n2slot, pslot = p, (p + 1) % 4, (p + 2) % 4, (p - 1) % 4

        @pl.when(g > 0)
        def _():
            _scatter_wait(pslot, 1 - rp)

        _gather_wait(slot, rp)

        def _scale(g2, _):
            nv = normb[slot, pl.ds(g2 * 16, 16)]
            for j in range(16):
                s = nv[j]
                e = g2 * 16 + j
                for k in range(D // 16):
                    rows[rp, e, pl.ds(k * 16, 16)] = (
                        rows[rp, e, pl.ds(k * 16, 16)] * s)
            return 0

        lax.fori_loop(0, C // 16, _scale, 0)

        _dst_wait(g, slot)
        pltpu.async_copy(rows.at[rp], agg_sh.at[dstb.at[slot]], ss[rp],
                         add=True)

        @pl.when(g + 1 < kc)
        def _():
            _aux_wait(g + 1, nslot)
            _gather_issue(nslot, 1 - rp)

        @pl.when(g + 2 < kc)
        def _():
            _aux_issue(g + 2, n2slot)

    def _quad(qq, _):
        for p in range(4):
            _iter(qq * 4 + p, p)
        return 0

    with _ns("edge_loop"):
        lax.fori_loop(0, kc // 4, _quad, 0)
        _scatter_wait(3, 1)

        plsc.subcore_barrier()

    with _ns("copy_out"):
        pltpu.sync_copy(agg_sh.at[pl.ds(base_row, rows_per_sub)],
                        out_hbm.at[cid, pl.ds(base_row, rows_per_sub)])


def _sc_agg(row_idx, dst, norm, hproj_flat, n_cores, n_subs):
    total_chunks = n_subs * (K0 + K1)
    assert total_chunks * C >= E
    ep = total_chunks * C
    pad = ep - E
    row_idx = jnp.concatenate([row_idx, jnp.zeros((pad,), jnp.int32)])
    dst = jnp.concatenate([dst, jnp.zeros((pad,), jnp.int32)])
    norm = jnp.concatenate([norm, jnp.zeros((pad,), jnp.float32)])
    rows_per_sub = ((-(-N // n_subs)) + 7) // 8 * 8
    n_pad = rows_per_sub * n_subs

    mesh = plsc.VectorSubcoreMesh(core_axis_name="c", subcore_axis_name="s")
    body = functools.partial(_sc_body, rows_per_sub=rows_per_sub)
    f = pl.kernel(
        body,
        out_type=jax.ShapeDtypeStruct((n_cores, n_pad, D), jnp.float32),
        mesh=mesh,
        scratch_types=[
            pltpu.VMEM_SHARED((n_pad, D), jnp.float32),
            pltpu.VMEM((4 * C,), jnp.int32),
            pltpu.VMEM((4, C), jnp.int32),
            pltpu.VMEM((4, C), jnp.float32),
            pltpu.VMEM((2, C, D), jnp.float32),
        ] + [pltpu.SemaphoreType.DMA] * 18,
    )
    return f(row_idx, dst, norm, hproj_flat)


def _combine_body(p_ref, feat_ref, lw_ref, out_ref):
    h = p_ref[...].sum(axis=0) + jnp.dot(feat_ref[...], lw_ref[...],
                                         preferred_element_type=jnp.float32)
    out_ref[...] = jnp.maximum(h, 0.0)


def _combine_tc(partials, feat, loop_weight):
    n_cores = partials.shape[0]
    return pl.pallas_call(
        _combine_body,
        grid=(NB,),
        in_specs=[
            pl.BlockSpec((n_cores, BN, D), lambda b: (0, b, 0)),
            pl.BlockSpec((BN, D), lambda b: (b, 0)),
            pl.BlockSpec((D, D), lambda b: (0, 0)),
        ],
        out_specs=pl.BlockSpec((BN, D), lambda b: (b, 0)),
        out_shape=jax.ShapeDtypeStruct((N, D), jnp.float32),
    )(partials, feat, loop_weight)


def kernel(feat, edge_index, etype, norm, v_b, a_rb, loop_weight):
    info = plsc.get_sparse_core_info()
    n_cores, n_subs = info.num_cores, info.num_subcores

    hproj = _hproj_tc(feat, v_b, a_rb)
    hproj_flat = hproj.reshape(R * N, D)

    src = edge_index[0]
    dst = edge_index[1]
    row_idx = etype * N + src
    partials = _sc_agg(row_idx, dst, norm[:, 0], hproj_flat, n_cores, n_subs)
    return _combine_tc(partials, feat, loop_weight)
